# baseline (speedup 1.0000x reference)
import functools

import jax
import jax.numpy as jnp
from jax import lax
from jax.experimental import pallas as pl
from jax.experimental.pallas import tpu as pltpu
from jax.experimental.pallas import tpu_sc as plsc

_BATCH = 16384
_FEAT_DIM = 64
_L = 16

_info = plsc.get_sparse_core_info()
_NC, _NS = _info.num_cores, _info.num_subcores
_NW = _NC * _NS
_B_PER_W = _BATCH // _NW
_GCHUNK = 128
_NCHUNK = _B_PER_W // _GCHUNK

_mesh = plsc.VectorSubcoreMesh(core_axis_name="c", subcore_axis_name="s")


@functools.partial(
    pl.kernel,
    mesh=_mesh,
    out_type=jax.ShapeDtypeStruct((_NW, _L), jnp.float32),
    scratch_types=[
        pltpu.VMEM((_NCHUNK, _GCHUNK), jnp.int32),
        pltpu.VMEM((_B_PER_W, _FEAT_DIM), jnp.float32),
        pltpu.VMEM((_B_PER_W, _FEAT_DIM), jnp.float32),
        pltpu.VMEM((_L,), jnp.float32),
        pltpu.SemaphoreType.DMA,
    ],
    compiler_params=pltpu.CompilerParams(use_tc_tiling_on_sc=False),
)
def _center_loss_partials(feats_hbm, labels_hbm, centers_hbm, out_hbm,
                          idx_v, rows_v, feats_v, acc_v, sem):
    wid = lax.axis_index("s") * _NC + lax.axis_index("c")
    base = wid * _B_PER_W

    pltpu.sync_copy(labels_hbm.at[pl.ds(wid * _NCHUNK, _NCHUNK)], idx_v)
    for j in range(_NCHUNK):
        pltpu.async_copy(centers_hbm.at[idx_v.at[j]],
                         rows_v.at[pl.ds(j * _GCHUNK, _GCHUNK)], sem)
    pltpu.sync_copy(feats_hbm.at[pl.ds(base, _B_PER_W)], feats_v)
    for j in range(_NCHUNK):
        pltpu.make_async_copy(centers_hbm.at[idx_v.at[j]],
                              rows_v.at[pl.ds(j * _GCHUNK, _GCHUNK)], sem).wait()

    def body(r, acc):
        for c in range(_FEAT_DIM // _L):
            d = feats_v[r, pl.ds(c * _L, _L)] - rows_v[r, pl.ds(c * _L, _L)]
            acc = acc + d * d
        return acc

    acc = lax.fori_loop(0, _B_PER_W, body, jnp.zeros((_L,), jnp.float32))
    acc_v[...] = acc
    pltpu.sync_copy(acc_v, out_hbm.at[wid])


def kernel(feats, labels, centers):
    labels_i32 = labels.astype(jnp.int32).reshape(_NW * _NCHUNK, _GCHUNK)
    partials = _center_loss_partials(feats, labels_i32, centers)
    return jnp.sum(partials) / (2.0 * _BATCH)


# speedup vs baseline: 1.6559x; 1.6559x over previous
import functools

import jax
import jax.numpy as jnp
from jax import lax
from jax.experimental import pallas as pl
from jax.experimental.pallas import tpu as pltpu
from jax.experimental.pallas import tpu_sc as plsc

_BATCH = 16384
_FEAT_DIM = 64
_L = 16

_info = plsc.get_sparse_core_info()
_NC, _NS = _info.num_cores, _info.num_subcores
_NW = _NC * _NS
_B_PER_W = _BATCH // _NW
_CHUNK = 64
_NCHUNK = _B_PER_W // _CHUNK

_mesh = plsc.VectorSubcoreMesh(core_axis_name="c", subcore_axis_name="s")


@functools.partial(
    pl.kernel,
    mesh=_mesh,
    out_type=jax.ShapeDtypeStruct((_NW, _L), jnp.float32),
    scratch_types=[
        pltpu.VMEM((_NCHUNK, _CHUNK), jnp.int32),
        pltpu.VMEM((_CHUNK, _FEAT_DIM), jnp.float32),
        pltpu.VMEM((_CHUNK, _FEAT_DIM), jnp.float32),
        pltpu.VMEM((_L,), jnp.float32),
        pltpu.SemaphoreType.DMA,
        pltpu.SemaphoreType.DMA,
    ],
)
def _center_loss_partials(feats_hbm, labels_hbm, centers_hbm, out_hbm,
                          idx_v, rows_v, feats_v, acc_v, sem, fsem):
    wid = lax.axis_index("s") * _NC + lax.axis_index("c")

    pltpu.sync_copy(labels_hbm.at[wid], idx_v)

    def step(c, acc):
        fcp = pltpu.async_copy(feats_hbm.at[wid, c], feats_v, fsem)
        copies = []
        for q in range(_CHUNK // _L):
            lbl_vec = idx_v[c, pl.ds(q * _L, _L)]
            for lane in range(_L):
                b = q * _L + lane
                copies.append(pltpu.async_copy(
                    centers_hbm.at[lbl_vec[lane]], rows_v.at[b], sem))
        fcp.wait()
        for cp in copies:
            cp.wait()

        def body(b, a):
            for cc in range(_FEAT_DIM // _L):
                d = (feats_v[b, pl.ds(cc * _L, _L)]
                     - rows_v[b, pl.ds(cc * _L, _L)])
                a = a + d * d
            return a

        return lax.fori_loop(0, _CHUNK, body, acc)

    acc = lax.fori_loop(0, _NCHUNK, step, jnp.zeros((_L,), jnp.float32))
    acc_v[...] = acc
    pltpu.sync_copy(acc_v, out_hbm.at[wid])


def kernel(feats, labels, centers):
    labels3 = labels.astype(jnp.int32).reshape(_NW, _NCHUNK, _CHUNK)
    feats4 = feats.reshape(_NW, _NCHUNK, _CHUNK, _FEAT_DIM)
    partials = _center_loss_partials(feats4, labels3, centers)
    return jnp.sum(partials) / (2.0 * _BATCH)
